# Initial kernel scaffold; baseline (speedup 1.0000x reference)
import jax
import jax.numpy as jnp
from jax.experimental import pallas as pl


def kernel(a):
    raise NotImplementedError("write your pallas kernel here")


# speedup vs baseline: 4.1340x; 4.1340x over previous
import functools

import jax
import jax.numpy as jnp
from jax.experimental import pallas as pl
from jax.experimental.pallas import tpu as pltpu

R = 256
N_ROWS = 8192
N_COLS = 2048


def _body(a_ref, o_ref, carry_ref):
    i = pl.program_id(0)

    @pl.when(i == 0)
    def _():
        carry_ref[...] = jnp.zeros_like(carry_ref)

    blk = a_ref[...]
    carry = carry_ref[...]
    rows = jax.lax.broadcasted_iota(jnp.int32, (R, R), 0)
    cols = jax.lax.broadcasted_iota(jnp.int32, (R, R), 1)
    strict_lower = (cols < rows).astype(jnp.float32)
    local_ex = jnp.dot(strict_lower, blk, preferred_element_type=jnp.float32)
    o_ref[...] = local_ex + carry
    carry_ref[...] = carry + jnp.sum(blk, axis=0, keepdims=True)


@jax.jit
def kernel(a):
    n_chunks = N_ROWS // R
    return pl.pallas_call(
        _body,
        grid=(n_chunks,),
        in_specs=[pl.BlockSpec((R, N_COLS), lambda i: (i, 0))],
        out_specs=pl.BlockSpec((R, N_COLS), lambda i: (i, 0)),
        out_shape=jax.ShapeDtypeStruct((N_ROWS, N_COLS), jnp.float32),
        scratch_shapes=[pltpu.VMEM((1, N_COLS), jnp.float32)],
        compiler_params=pltpu.CompilerParams(
            dimension_semantics=("arbitrary",),
        ),
    )(a)


# speedup vs baseline: 4.5505x; 1.1007x over previous
import functools

import jax
import jax.numpy as jnp
from jax.experimental import pallas as pl
from jax.experimental.pallas import tpu as pltpu

R = 512
N_ROWS = 8192
N_COLS = 2048


def _body(a_ref, o_ref, carry_ref):
    i = pl.program_id(0)

    @pl.when(i == 0)
    def _():
        carry_ref[...] = jnp.zeros_like(carry_ref)

    blk = a_ref[...]
    carry = carry_ref[...]
    rows = jax.lax.broadcasted_iota(jnp.int32, (R, R), 0)
    cols = jax.lax.broadcasted_iota(jnp.int32, (R, R), 1)
    strict_lower = (cols < rows).astype(jnp.float32)
    local_ex = jnp.dot(strict_lower, blk, preferred_element_type=jnp.float32)
    o_ref[...] = local_ex + carry
    carry_ref[...] = carry + jnp.sum(blk, axis=0, keepdims=True)


@jax.jit
def kernel(a):
    n_chunks = N_ROWS // R
    return pl.pallas_call(
        _body,
        grid=(n_chunks,),
        in_specs=[pl.BlockSpec((R, N_COLS), lambda i: (i, 0))],
        out_specs=pl.BlockSpec((R, N_COLS), lambda i: (i, 0)),
        out_shape=jax.ShapeDtypeStruct((N_ROWS, N_COLS), jnp.float32),
        scratch_shapes=[pltpu.VMEM((1, N_COLS), jnp.float32)],
        compiler_params=pltpu.CompilerParams(
            dimension_semantics=("arbitrary",),
        ),
    )(a)
